# Initial kernel scaffold; baseline (speedup 1.0000x reference)
import jax
import jax.numpy as jnp
from jax.experimental import pallas as pl


def kernel(x, Wg, bg, We, be):
    raise NotImplementedError("write your pallas kernel here")


# speedup vs baseline: 1.9279x; 1.9279x over previous
import jax
import jax.numpy as jnp
from jax import lax
from jax.experimental import pallas as pl
from jax.experimental.pallas import tpu as pltpu

T = 4096
D_IN = 1024
D_OUT = 1024
E = 8
K = 2
BT = 512


def _moe_body(x_ref, wg_ref, bg_ref, we_ref, be_ref, out_ref, ew_ref):
    e = pl.program_id(1)
    x = x_ref[...]

    logits = jnp.dot(x, wg_ref[...], preferred_element_type=jnp.float32)
    logits = logits + bg_ref[...]
    iota = lax.broadcasted_iota(jnp.int32, (BT, E), 1)
    m1 = jnp.max(logits, axis=-1, keepdims=True)
    i1 = jnp.min(jnp.where(logits == m1, iota, E), axis=-1, keepdims=True)
    mask1 = iota == i1
    l2 = jnp.where(mask1, -jnp.inf, logits)
    m2 = jnp.max(l2, axis=-1, keepdims=True)
    i2 = jnp.min(jnp.where(l2 == m2, iota, E), axis=-1, keepdims=True)
    mask2 = iota == i2
    b = jnp.exp(m2 - m1)
    denom = 1.0 + b
    w = jnp.where(mask1, 1.0 / denom, 0.0) + jnp.where(mask2, b / denom, 0.0)

    @pl.when(e == 0)
    def _():
        ew_ref[...] = w

    w_col = jnp.sum(jnp.where(iota == e, w, 0.0), axis=-1, keepdims=True)
    contrib = (jnp.dot(x, we_ref[0], preferred_element_type=jnp.float32)
               + be_ref[0]) * w_col

    @pl.when(e == 0)
    def _():
        out_ref[...] = contrib

    @pl.when(e > 0)
    def _():
        out_ref[...] += contrib


def kernel(x, Wg, bg, We, be):
    bg2 = bg.reshape(1, E)
    be3 = be.reshape(E, 1, D_OUT)
    grid = (T // BT, E)
    out, ew = pl.pallas_call(
        _moe_body,
        grid=grid,
        in_specs=[
            pl.BlockSpec((BT, D_IN), lambda i, e: (i, 0)),
            pl.BlockSpec((D_IN, E), lambda i, e: (0, 0)),
            pl.BlockSpec((1, E), lambda i, e: (0, 0)),
            pl.BlockSpec((1, D_IN, D_OUT), lambda i, e: (e, 0, 0)),
            pl.BlockSpec((1, 1, D_OUT), lambda i, e: (e, 0, 0)),
        ],
        out_specs=[
            pl.BlockSpec((BT, D_OUT), lambda i, e: (i, 0)),
            pl.BlockSpec((BT, E), lambda i, e: (i, 0)),
        ],
        out_shape=[
            jax.ShapeDtypeStruct((T, D_OUT), jnp.float32),
            jax.ShapeDtypeStruct((T, E), jnp.float32),
        ],
        compiler_params=pltpu.CompilerParams(
            dimension_semantics=("parallel", "arbitrary"),
        ),
    )(x, Wg, bg2, We, be3)
    return out, ew


# speedup vs baseline: 1.9284x; 1.0002x over previous
import jax
import jax.numpy as jnp
from jax import lax
from jax.experimental import pallas as pl
from jax.experimental.pallas import tpu as pltpu

T = 4096
D_IN = 1024
D_OUT = 1024
E = 8
K = 2
BT = 512


def _moe_body(x_ref, wg_ref, bg_ref, we_ref, be_ref, out_ref, ew_ref):
    e = pl.program_id(1)
    x = x_ref[...]

    logits = jnp.dot(x, wg_ref[...], preferred_element_type=jnp.float32)
    logits = logits + bg_ref[...]
    iota = lax.broadcasted_iota(jnp.int32, (BT, E), 1)
    m1 = jnp.max(logits, axis=-1, keepdims=True)
    i1 = jnp.min(jnp.where(logits == m1, iota, E), axis=-1, keepdims=True)
    mask1 = iota == i1
    l2 = jnp.where(mask1, -jnp.inf, logits)
    m2 = jnp.max(l2, axis=-1, keepdims=True)
    i2 = jnp.min(jnp.where(l2 == m2, iota, E), axis=-1, keepdims=True)
    mask2 = iota == i2
    b = jnp.exp(m2 - m1)
    denom = 1.0 + b
    w = jnp.where(mask1, 1.0 / denom, 0.0) + jnp.where(mask2, b / denom, 0.0)

    @pl.when(e == 0)
    def _():
        ew_ref[...] = w

    w_col = jnp.sum(jnp.where(iota == e, w, 0.0), axis=-1, keepdims=True)
    contrib = (jnp.dot(x.astype(jnp.bfloat16), we_ref[0].astype(jnp.bfloat16),
                       preferred_element_type=jnp.float32)
               + be_ref[0]) * w_col

    @pl.when(e == 0)
    def _():
        out_ref[...] = contrib

    @pl.when(e > 0)
    def _():
        out_ref[...] += contrib


def kernel(x, Wg, bg, We, be):
    bg2 = bg.reshape(1, E)
    be3 = be.reshape(E, 1, D_OUT)
    grid = (T // BT, E)
    out, ew = pl.pallas_call(
        _moe_body,
        grid=grid,
        in_specs=[
            pl.BlockSpec((BT, D_IN), lambda i, e: (i, 0)),
            pl.BlockSpec((D_IN, E), lambda i, e: (0, 0)),
            pl.BlockSpec((1, E), lambda i, e: (0, 0)),
            pl.BlockSpec((1, D_IN, D_OUT), lambda i, e: (e, 0, 0)),
            pl.BlockSpec((1, 1, D_OUT), lambda i, e: (e, 0, 0)),
        ],
        out_specs=[
            pl.BlockSpec((BT, D_OUT), lambda i, e: (i, 0)),
            pl.BlockSpec((BT, E), lambda i, e: (i, 0)),
        ],
        out_shape=[
            jax.ShapeDtypeStruct((T, D_OUT), jnp.float32),
            jax.ShapeDtypeStruct((T, E), jnp.float32),
        ],
        compiler_params=pltpu.CompilerParams(
            dimension_semantics=("parallel", "arbitrary"),
        ),
    )(x, Wg, bg2, We, be3)
    return out, ew


# speedup vs baseline: 2.2948x; 1.1900x over previous
import jax
import jax.numpy as jnp
from jax import lax
from jax.experimental import pallas as pl
from jax.experimental.pallas import tpu as pltpu

T = 4096
D_IN = 1024
D_OUT = 1024
E = 8
K = 2
BT = 1024


def _moe_body(x_ref, wg_ref, bg_ref, we_ref, be_ref, out_ref, ew_ref):
    e = pl.program_id(1)
    x = x_ref[...]

    logits = jnp.dot(x, wg_ref[...], preferred_element_type=jnp.float32)
    logits = logits + bg_ref[...]
    iota = lax.broadcasted_iota(jnp.int32, (BT, E), 1)
    m1 = jnp.max(logits, axis=-1, keepdims=True)
    i1 = jnp.min(jnp.where(logits == m1, iota, E), axis=-1, keepdims=True)
    mask1 = iota == i1
    l2 = jnp.where(mask1, -jnp.inf, logits)
    m2 = jnp.max(l2, axis=-1, keepdims=True)
    i2 = jnp.min(jnp.where(l2 == m2, iota, E), axis=-1, keepdims=True)
    mask2 = iota == i2
    b = jnp.exp(m2 - m1)
    denom = 1.0 + b
    w = jnp.where(mask1, 1.0 / denom, 0.0) + jnp.where(mask2, b / denom, 0.0)

    @pl.when(e == 0)
    def _():
        ew_ref[...] = w

    w_col = jnp.sum(jnp.where(iota == e, w, 0.0), axis=-1, keepdims=True)
    contrib = (jnp.dot(x.astype(jnp.bfloat16), we_ref[0].astype(jnp.bfloat16),
                       preferred_element_type=jnp.float32)
               + be_ref[0]) * w_col

    @pl.when(e == 0)
    def _():
        out_ref[...] = contrib

    @pl.when(e > 0)
    def _():
        out_ref[...] += contrib


def kernel(x, Wg, bg, We, be):
    bg2 = bg.reshape(1, E)
    be3 = be.reshape(E, 1, D_OUT)
    grid = (T // BT, E)
    out, ew = pl.pallas_call(
        _moe_body,
        grid=grid,
        in_specs=[
            pl.BlockSpec((BT, D_IN), lambda i, e: (i, 0)),
            pl.BlockSpec((D_IN, E), lambda i, e: (0, 0)),
            pl.BlockSpec((1, E), lambda i, e: (0, 0)),
            pl.BlockSpec((1, D_IN, D_OUT), lambda i, e: (e, 0, 0)),
            pl.BlockSpec((1, 1, D_OUT), lambda i, e: (e, 0, 0)),
        ],
        out_specs=[
            pl.BlockSpec((BT, D_OUT), lambda i, e: (i, 0)),
            pl.BlockSpec((BT, E), lambda i, e: (i, 0)),
        ],
        out_shape=[
            jax.ShapeDtypeStruct((T, D_OUT), jnp.float32),
            jax.ShapeDtypeStruct((T, E), jnp.float32),
        ],
        compiler_params=pltpu.CompilerParams(
            dimension_semantics=("parallel", "arbitrary"),
        ),
    )(x, Wg, bg2, We, be3)
    return out, ew


# speedup vs baseline: 2.3818x; 1.0379x over previous
import jax
import jax.numpy as jnp
from jax import lax
from jax.experimental import pallas as pl
from jax.experimental.pallas import tpu as pltpu

T = 4096
D_IN = 1024
D_OUT = 1024
E = 8
K = 2
BT = 2048


def _moe_body(x_ref, wg_ref, bg_ref, we_ref, be_ref, out_ref, ew_ref):
    e = pl.program_id(1)
    x = x_ref[...]

    logits = jnp.dot(x, wg_ref[...], preferred_element_type=jnp.float32)
    logits = logits + bg_ref[...]
    iota = lax.broadcasted_iota(jnp.int32, (BT, E), 1)
    m1 = jnp.max(logits, axis=-1, keepdims=True)
    i1 = jnp.min(jnp.where(logits == m1, iota, E), axis=-1, keepdims=True)
    mask1 = iota == i1
    l2 = jnp.where(mask1, -jnp.inf, logits)
    m2 = jnp.max(l2, axis=-1, keepdims=True)
    i2 = jnp.min(jnp.where(l2 == m2, iota, E), axis=-1, keepdims=True)
    mask2 = iota == i2
    b = jnp.exp(m2 - m1)
    denom = 1.0 + b
    w = jnp.where(mask1, 1.0 / denom, 0.0) + jnp.where(mask2, b / denom, 0.0)

    @pl.when(e == 0)
    def _():
        ew_ref[...] = w

    w_col = jnp.sum(jnp.where(iota == e, w, 0.0), axis=-1, keepdims=True)
    contrib = (jnp.dot(x.astype(jnp.bfloat16), we_ref[0].astype(jnp.bfloat16),
                       preferred_element_type=jnp.float32)
               + be_ref[0]) * w_col

    @pl.when(e == 0)
    def _():
        out_ref[...] = contrib

    @pl.when(e > 0)
    def _():
        out_ref[...] += contrib


def kernel(x, Wg, bg, We, be):
    bg2 = bg.reshape(1, E)
    be3 = be.reshape(E, 1, D_OUT)
    grid = (T // BT, E)
    out, ew = pl.pallas_call(
        _moe_body,
        grid=grid,
        in_specs=[
            pl.BlockSpec((BT, D_IN), lambda i, e: (i, 0)),
            pl.BlockSpec((D_IN, E), lambda i, e: (0, 0)),
            pl.BlockSpec((1, E), lambda i, e: (0, 0)),
            pl.BlockSpec((1, D_IN, D_OUT), lambda i, e: (e, 0, 0)),
            pl.BlockSpec((1, 1, D_OUT), lambda i, e: (e, 0, 0)),
        ],
        out_specs=[
            pl.BlockSpec((BT, D_OUT), lambda i, e: (i, 0)),
            pl.BlockSpec((BT, E), lambda i, e: (i, 0)),
        ],
        out_shape=[
            jax.ShapeDtypeStruct((T, D_OUT), jnp.float32),
            jax.ShapeDtypeStruct((T, E), jnp.float32),
        ],
        compiler_params=pltpu.CompilerParams(
            dimension_semantics=("parallel", "arbitrary"),
        ),
    )(x, Wg, bg2, We, be3)
    return out, ew
